# Initial kernel scaffold; baseline (speedup 1.0000x reference)
import jax
import jax.numpy as jnp
from jax.experimental import pallas as pl


def kernel(x, edge_index, params):
    raise NotImplementedError("write your pallas kernel here")


# speedup vs baseline: 5.0857x; 5.0857x over previous
import functools

import jax
import jax.numpy as jnp
from jax import lax
from jax.experimental import pallas as pl
from jax.experimental.pallas import tpu as pltpu
from jax.experimental.pallas import tpu_sc as plsc

N = 10000
E = 320000
D = 128
H = 8
NHID = H * D
NTILES = 16
NSC = 2
E_TOT = E + N
GROUP = 64
EPT = ((E_TOT + NTILES * GROUP - 1) // (NTILES * GROUP)) * GROUP
E_PAD = EPT * NTILES
NGROUPS = EPT // GROUP
STRIPE = 624
TAIL = N - NTILES * STRIPE
CH = 16
NCH = STRIPE // CH
RB = 1000
NRB = N // RB

_f32 = jnp.float32


def _lrelu(x, slope):
    return jnp.maximum(x, slope * x)


def _ln(y, g, b):
    m = jnp.mean(y, axis=-1, keepdims=True)
    v = jnp.mean((y - m) ** 2, axis=-1, keepdims=True)
    return (y - m) * lax.rsqrt(v + 1e-5) * g + b


def _embed_body(x_ref, w1_ref, b1_ref, w2_ref, b2_ref, o_ref):
    h1 = _lrelu(jnp.dot(x_ref[...], w1_ref[...],
                        preferred_element_type=_f32) + b1_ref[...], 0.01)
    o_ref[...] = jnp.dot(h1, w2_ref[...],
                         preferred_element_type=_f32) + b2_ref[...]


def _embed(x, w1, b1, w2, b2):
    return pl.pallas_call(
        _embed_body,
        grid=(NRB,),
        in_specs=[
            pl.BlockSpec((RB, D), lambda i: (i, 0)),
            pl.BlockSpec((D, D), lambda i: (0, 0)),
            pl.BlockSpec((1, D), lambda i: (0, 0)),
            pl.BlockSpec((D, D), lambda i: (0, 0)),
            pl.BlockSpec((1, D), lambda i: (0, 0)),
        ],
        out_specs=pl.BlockSpec((RB, D), lambda i: (i, 0)),
        out_shape=jax.ShapeDtypeStruct((N, D), _f32),
    )(x, w1, b1[None, :], w2, b2[None, :])


def _xlxr_body(h_ref, wl_ref, bl_ref, wr_ref, br_ref, att_ref,
               xl_ref, xr_ref, m_ref, sm):
    i = pl.program_id(1)
    xlb = jnp.dot(h_ref[...], wl_ref[...],
                  preferred_element_type=_f32) + bl_ref[0]
    xrb = jnp.dot(h_ref[...], wr_ref[...],
                  preferred_element_type=_f32) + br_ref[0]
    xl_ref[...] = xlb
    xr_ref[...] = xrb
    attr = jnp.abs(att_ref[0])
    pmax = jnp.max(jnp.sum(jnp.abs(xlb) * attr, axis=1))
    qmax = jnp.max(jnp.sum(jnp.abs(xrb) * attr, axis=1))

    @pl.when(i == 0)
    def _():
        sm[0] = pmax
        sm[1] = qmax

    @pl.when(i > 0)
    def _():
        sm[0] = jnp.maximum(sm[0], pmax)
        sm[1] = jnp.maximum(sm[1], qmax)

    m_ref[...] = jnp.full((1, 1, 16), sm[0] + sm[1], _f32)


def _xlxr(h, wl, bl, wr, br, att):
    return pl.pallas_call(
        _xlxr_body,
        grid=(H, NRB),
        in_specs=[
            pl.BlockSpec((RB, D), lambda j, i: (i, 0)),
            pl.BlockSpec((D, D), lambda j, i: (0, j)),
            pl.BlockSpec((1, 1, D), lambda j, i: (j, 0, 0)),
            pl.BlockSpec((D, D), lambda j, i: (0, j)),
            pl.BlockSpec((1, 1, D), lambda j, i: (j, 0, 0)),
            pl.BlockSpec((1, 1, D), lambda j, i: (j, 0, 0)),
        ],
        out_specs=[
            pl.BlockSpec((RB, D), lambda j, i: (j * NRB + i, 0)),
            pl.BlockSpec((RB, D), lambda j, i: (j * NRB + i, 0)),
            pl.BlockSpec((1, 1, 16), lambda j, i: (j, 0, 0)),
        ],
        out_shape=[
            jax.ShapeDtypeStruct((H * N, D), _f32),
            jax.ShapeDtypeStruct((H * N, D), _f32),
            jax.ShapeDtypeStruct((H, 1, 16), _f32),
        ],
        scratch_shapes=[pltpu.SMEM((2,), _f32)],
    )(h, wl, bl.reshape(H, 1, D), wr, br.reshape(H, 1, D),
      att.reshape(H, 1, D))


def _edge_body(xl_hbm, xr_hbm, src_hbm, dst_hbm, att_hbm, mb_hbm,
               outn_hbm, outd_hbm,
               acc_n, acc_d, buf_l, buf_r, wrow,
               rsrc, rdst, gsrc, gdst, attbuf, mbbuf, red,
               tmp_n, tmp_d, ttn, ttd, zidx, tidx, sem):
    c = lax.axis_index("c")
    s = lax.axis_index("s")
    tbase = s * EPT

    pltpu.sync_copy(att_hbm, attbuf)
    pltpu.sync_copy(mb_hbm, mbbuf)

    zv = jnp.zeros((16,), _f32)
    iot = lax.iota(jnp.int32, 16)
    tidx[...] = iot + (NTILES * STRIPE)

    def head_step(hh, _):
        h = c * 4 + hh
        hoff = h * N
        attv = [attbuf[pl.ds(h * D + 16 * j, 16)] for j in range(8)]
        mbv = mbbuf[pl.ds(h * 16, 16)]

        def zero_row(r, _):
            for j in range(8):
                tmp_n[r, pl.ds(16 * j, 16)] = zv
            tmp_d[r, pl.ds(0, 16)] = zv
            return 0

        lax.fori_loop(0, CH, zero_row, 0, unroll=False)
        for k in range(NCH):
            for t in range(CH // 16):
                zidx[pl.ds(16 * t, 16)] = iot + (
                    s * STRIPE + CH * k + 16 * t)
            pltpu.sync_copy(tmp_n, acc_n.at[zidx])
            pltpu.sync_copy(tmp_d, acc_d.at[zidx])

        @pl.when(s == NTILES - 1)
        def _():
            for t in range(TAIL):
                for j in range(8):
                    ttn[t, pl.ds(16 * j, 16)] = zv
                ttd[t, pl.ds(0, 16)] = zv
            pltpu.sync_copy(ttn, acc_n.at[tidx])
            pltpu.sync_copy(ttd, acc_d.at[tidx])

        plsc.subcore_barrier()

        def group_step(g, _):
            pltpu.sync_copy(src_hbm.at[pl.ds(tbase + g * GROUP, GROUP)], rsrc)
            pltpu.sync_copy(dst_hbm.at[pl.ds(tbase + g * GROUP, GROUP)], rdst)
            for k in range(GROUP // 16):
                gsrc[pl.ds(16 * k, 16)] = rsrc[pl.ds(16 * k, 16)] + hoff
                gdst[pl.ds(16 * k, 16)] = rdst[pl.ds(16 * k, 16)] + hoff
            pltpu.async_copy(xl_hbm.at[gsrc], buf_l, sem).wait()
            pltpu.async_copy(xr_hbm.at[gdst], buf_r, sem).wait()

            def sub_step(u, _):
                for i in range(16):
                    e = u * 16 + i
                    acc = jnp.zeros((16,), _f32)
                    for j in range(8):
                        a = buf_l[e, pl.ds(16 * j, 16)]
                        b = buf_r[e, pl.ds(16 * j, 16)]
                        t = a + b
                        t = jnp.maximum(t, 0.2 * t)
                        acc = acc + attv[j] * t
                    for sh in (1, 2, 4, 8):
                        perm = lax.iota(jnp.int32, 16) ^ sh
                        red[...] = acc
                        acc = acc + plsc.load_gather(red, [perm])
                    gid = tbase + g * GROUP + e
                    maskf = jnp.where(gid < E_TOT, 1.0, 0.0)
                    wf = jnp.exp(acc - mbv) * maskf
                    wrow[e, pl.ds(0, 16)] = wf
                    for j in range(8):
                        buf_l[e, pl.ds(16 * j, 16)] = (
                            wf * buf_l[e, pl.ds(16 * j, 16)])
                return 0

            lax.fori_loop(0, GROUP // 16, sub_step, 0, unroll=False)
            pltpu.sync_copy(buf_l, acc_n.at[rdst], add=True)
            pltpu.sync_copy(wrow, acc_d.at[rdst], add=True)
            return 0

        lax.fori_loop(0, NGROUPS, group_step, 0, unroll=False)
        plsc.subcore_barrier()

        for k in range(NCH):
            for t in range(CH // 16):
                zidx[pl.ds(16 * t, 16)] = iot + (
                    s * STRIPE + CH * k + 16 * t)
            pltpu.async_copy(acc_n.at[zidx], tmp_n, sem).wait()
            pltpu.sync_copy(
                tmp_n, outn_hbm.at[pl.ds(h * N + s * STRIPE + CH * k, CH)])
            pltpu.async_copy(acc_d.at[zidx], tmp_d, sem).wait()
            pltpu.sync_copy(
                tmp_d, outd_hbm.at[pl.ds(h * N + s * STRIPE + CH * k, CH)])

        @pl.when(s == NTILES - 1)
        def _():
            pltpu.async_copy(acc_n.at[tidx], ttn, sem).wait()
            pltpu.sync_copy(
                ttn, outn_hbm.at[pl.ds(h * N + NTILES * STRIPE, TAIL)])
            pltpu.async_copy(acc_d.at[tidx], ttd, sem).wait()
            pltpu.sync_copy(
                ttd, outd_hbm.at[pl.ds(h * N + NTILES * STRIPE, TAIL)])

        return 0

    lax.fori_loop(0, 4, head_step, 0, unroll=False)


def _edge_phase(xl, xr, src_pad, dst_pad, att_flat, mb_flat):
    mesh = plsc.VectorSubcoreMesh(core_axis_name="c", subcore_axis_name="s")
    f = pl.kernel(
        _edge_body,
        out_type=[
            jax.ShapeDtypeStruct((H * N, D), _f32),
            jax.ShapeDtypeStruct((H * N, 16), _f32),
        ],
        mesh=mesh,
        scratch_types=[
            pltpu.VMEM_SHARED((N, D), _f32),
            pltpu.VMEM_SHARED((N, 16), _f32),
            pltpu.VMEM((GROUP, D), _f32),
            pltpu.VMEM((GROUP, D), _f32),
            pltpu.VMEM((GROUP, 16), _f32),
            pltpu.VMEM((GROUP,), jnp.int32),
            pltpu.VMEM((GROUP,), jnp.int32),
            pltpu.VMEM((GROUP,), jnp.int32),
            pltpu.VMEM((GROUP,), jnp.int32),
            pltpu.VMEM((NHID,), _f32),
            pltpu.VMEM((H * 16,), _f32),
            pltpu.VMEM((16,), _f32),
            pltpu.VMEM((CH, D), _f32),
            pltpu.VMEM((CH, 16), _f32),
            pltpu.VMEM((TAIL, D), _f32),
            pltpu.VMEM((TAIL, 16), _f32),
            pltpu.VMEM((CH,), jnp.int32),
            pltpu.VMEM((TAIL,), jnp.int32),
            pltpu.SemaphoreType.DMA,
        ],
        compiler_params=pltpu.CompilerParams(needs_layout_passes=False,
                                             use_tc_tiling_on_sc=False),
    )
    return f(xl, xr, src_pad, dst_pad, att_flat, mb_flat)


def _proj_body(num_ref, den_ref, gb_ref, pw_ref, pb_ref, res_ref,
               g_ref, b_ref, o_ref, acc):
    j = pl.program_id(1)
    gat = num_ref[...] / (den_ref[:, 0:1] + 1e-16) + gb_ref[0]
    contrib = jnp.dot(gat, pw_ref[...], preferred_element_type=_f32)

    @pl.when(j == 0)
    def _():
        acc[...] = contrib

    @pl.when(j > 0)
    def _():
        acc[...] = acc[...] + contrib

    @pl.when(j == H - 1)
    def _():
        y = acc[...] + pb_ref[...] + res_ref[...]
        o_ref[...] = _ln(y, g_ref[...], b_ref[...])


def _proj_ln(num, den, gat_bias, pw, pb, res, g, b):
    return pl.pallas_call(
        _proj_body,
        grid=(NRB, H),
        in_specs=[
            pl.BlockSpec((RB, D), lambda i, j: (j * NRB + i, 0)),
            pl.BlockSpec((RB, 16), lambda i, j: (j * NRB + i, 0)),
            pl.BlockSpec((1, 1, D), lambda i, j: (j, 0, 0)),
            pl.BlockSpec((D, D), lambda i, j: (j, 0)),
            pl.BlockSpec((1, D), lambda i, j: (0, 0)),
            pl.BlockSpec((RB, D), lambda i, j: (i, 0)),
            pl.BlockSpec((1, D), lambda i, j: (0, 0)),
            pl.BlockSpec((1, D), lambda i, j: (0, 0)),
        ],
        out_specs=pl.BlockSpec((RB, D), lambda i, j: (i, 0)),
        out_shape=jax.ShapeDtypeStruct((N, D), _f32),
        scratch_shapes=[pltpu.VMEM((RB, D), _f32)],
    )(num, den, gat_bias.reshape(H, 1, D), pw, pb[None, :],
      res, g[None, :], b[None, :])


def _fc_body(h_ref, w1_ref, b1_ref, w2_ref, b2_ref, g_ref, b_ref, o_ref):
    hb = h_ref[...]
    f = _lrelu(jnp.dot(hb, w1_ref[...],
                       preferred_element_type=_f32) + b1_ref[...], 0.01)
    f = jnp.dot(f, w2_ref[...], preferred_element_type=_f32) + b2_ref[...]
    o_ref[...] = _ln(f + hb, g_ref[...], b_ref[...])


def _fc_ln(h, w1, b1, w2, b2, g, b):
    return pl.pallas_call(
        _fc_body,
        grid=(NRB,),
        in_specs=[
            pl.BlockSpec((RB, D), lambda i: (i, 0)),
            pl.BlockSpec((D, D), lambda i: (0, 0)),
            pl.BlockSpec((1, D), lambda i: (0, 0)),
            pl.BlockSpec((D, D), lambda i: (0, 0)),
            pl.BlockSpec((1, D), lambda i: (0, 0)),
            pl.BlockSpec((1, D), lambda i: (0, 0)),
            pl.BlockSpec((1, D), lambda i: (0, 0)),
        ],
        out_specs=pl.BlockSpec((RB, D), lambda i: (i, 0)),
        out_shape=jax.ShapeDtypeStruct((N, D), _f32),
    )(h, w1, b1[None, :], w2, b2[None, :], g[None, :], b[None, :])


def kernel(x, edge_index, params):
    p = params
    loops = jnp.arange(N, dtype=edge_index.dtype)
    padv = jnp.zeros((E_PAD - E_TOT,), edge_index.dtype)
    src_pad = jnp.concatenate([edge_index[0], loops, padv])
    dst_pad = jnp.concatenate([edge_index[1], loops, padv])

    h = _embed(x, p['emb_W1'], p['emb_b1'], p['emb_W2'], p['emb_b2'])
    for i in range(2):
        xl, xr, mb = _xlxr(h, p['gat%d_Wl' % i], p['gat%d_bl' % i],
                           p['gat%d_Wr' % i], p['gat%d_br' % i],
                           p['gat%d_att' % i])
        num, den = _edge_phase(xl, xr, src_pad, dst_pad,
                               p['gat%d_att' % i].reshape(-1),
                               mb.reshape(-1))
        h = _proj_ln(num, den, p['gat%d_bias' % i],
                     p['proj%d_W' % i], p['proj%d_b' % i], h,
                     p['gn%d_g' % i], p['gn%d_b' % i])
        h = _fc_ln(h, p['fc%d_W1' % i], p['fc%d_b1' % i],
                   p['fc%d_W2' % i], p['fc%d_b2' % i],
                   p['fn%d_g' % i], p['fn%d_b' % i])
    return h


# speedup vs baseline: 6.7976x; 1.3366x over previous
import functools

import jax
import jax.numpy as jnp
from jax import lax
from jax.experimental import pallas as pl
from jax.experimental.pallas import tpu as pltpu
from jax.experimental.pallas import tpu_sc as plsc

N = 10000
E = 320000
D = 128
H = 8
NHID = H * D
NTILES = 16
NSC = 2
E_TOT = E + N
GROUP = 64
NGROUPS = -(-E_TOT // (NTILES * GROUP))
NGROUPS += NGROUPS % 2
EPT = NGROUPS * GROUP
E_PAD = EPT * NTILES
STRIPE = 624
TAIL = N - NTILES * STRIPE
CH = 16
NCH = STRIPE // CH
RB = 1000
NRB = N // RB

_f32 = jnp.float32


def _lrelu(x, slope):
    return jnp.maximum(x, slope * x)


def _ln(y, g, b):
    m = jnp.mean(y, axis=-1, keepdims=True)
    v = jnp.mean((y - m) ** 2, axis=-1, keepdims=True)
    return (y - m) * lax.rsqrt(v + 1e-5) * g + b


def _embed_body(x_ref, w1_ref, b1_ref, w2_ref, b2_ref, o_ref):
    h1 = _lrelu(jnp.dot(x_ref[...], w1_ref[...],
                        preferred_element_type=_f32) + b1_ref[...], 0.01)
    o_ref[...] = jnp.dot(h1, w2_ref[...],
                         preferred_element_type=_f32) + b2_ref[...]


def _embed(x, w1, b1, w2, b2):
    return pl.pallas_call(
        _embed_body,
        grid=(NRB,),
        in_specs=[
            pl.BlockSpec((RB, D), lambda i: (i, 0)),
            pl.BlockSpec((D, D), lambda i: (0, 0)),
            pl.BlockSpec((1, D), lambda i: (0, 0)),
            pl.BlockSpec((D, D), lambda i: (0, 0)),
            pl.BlockSpec((1, D), lambda i: (0, 0)),
        ],
        out_specs=pl.BlockSpec((RB, D), lambda i: (i, 0)),
        out_shape=jax.ShapeDtypeStruct((N, D), _f32),
    )(x, w1, b1[None, :], w2, b2[None, :])


def _xlxr_body(h_ref, wl_ref, bl_ref, wr_ref, br_ref, att_ref,
               xl_ref, xr_ref, m_ref, sm):
    i = pl.program_id(1)
    xlb = jnp.dot(h_ref[...], wl_ref[...],
                  preferred_element_type=_f32) + bl_ref[0]
    xrb = jnp.dot(h_ref[...], wr_ref[...],
                  preferred_element_type=_f32) + br_ref[0]
    xl_ref[...] = xlb
    xr_ref[...] = xrb
    attr = jnp.abs(att_ref[0])
    pmax = jnp.max(jnp.sum(jnp.abs(xlb) * attr, axis=1))
    qmax = jnp.max(jnp.sum(jnp.abs(xrb) * attr, axis=1))

    @pl.when(i == 0)
    def _():
        sm[0] = pmax
        sm[1] = qmax

    @pl.when(i > 0)
    def _():
        sm[0] = jnp.maximum(sm[0], pmax)
        sm[1] = jnp.maximum(sm[1], qmax)

    m_ref[...] = jnp.full((1, 1, 16), sm[0] + sm[1], _f32)


def _xlxr(h, wl, bl, wr, br, att):
    return pl.pallas_call(
        _xlxr_body,
        grid=(H, NRB),
        in_specs=[
            pl.BlockSpec((RB, D), lambda j, i: (i, 0)),
            pl.BlockSpec((D, D), lambda j, i: (0, j)),
            pl.BlockSpec((1, 1, D), lambda j, i: (j, 0, 0)),
            pl.BlockSpec((D, D), lambda j, i: (0, j)),
            pl.BlockSpec((1, 1, D), lambda j, i: (j, 0, 0)),
            pl.BlockSpec((1, 1, D), lambda j, i: (j, 0, 0)),
        ],
        out_specs=[
            pl.BlockSpec((RB, D), lambda j, i: (j * NRB + i, 0)),
            pl.BlockSpec((RB, D), lambda j, i: (j * NRB + i, 0)),
            pl.BlockSpec((1, 1, 16), lambda j, i: (j, 0, 0)),
        ],
        out_shape=[
            jax.ShapeDtypeStruct((H * N, D), _f32),
            jax.ShapeDtypeStruct((H * N, D), _f32),
            jax.ShapeDtypeStruct((H, 1, 16), _f32),
        ],
        scratch_shapes=[pltpu.SMEM((2,), _f32)],
    )(h, wl, bl.reshape(H, 1, D), wr, br.reshape(H, 1, D),
      att.reshape(H, 1, D))


def _edge_body(xl_hbm, xr_hbm, src_hbm, dst_hbm, att_hbm, mb_hbm,
               outn_hbm, outd_hbm,
               acc_n, acc_d, buf_l0, buf_l1, buf_r0, buf_r1, wrow,
               rsrc, rdst, gsrc0, gsrc1, gdst0, gdst1, sdst0, sdst1,
               attbuf, mbbuf, red, tmp_n, tmp_d, zidx, tidx,
               sem, seml0, seml1, semr0, semr1):
    c = lax.axis_index("c")
    s = lax.axis_index("s")
    tbase = s * EPT
    buf_l = (buf_l0, buf_l1)
    buf_r = (buf_r0, buf_r1)
    gsrc = (gsrc0, gsrc1)
    gdst = (gdst0, gdst1)
    sdst = (sdst0, sdst1)
    seml = (seml0, seml1)
    semr = (semr0, semr1)

    pltpu.sync_copy(att_hbm, attbuf)
    pltpu.sync_copy(mb_hbm, mbbuf)

    zv = jnp.zeros((16,), _f32)
    iot = lax.iota(jnp.int32, 16)
    tidx[...] = iot + (NTILES * STRIPE)

    def head_step(hh, _):
        h = c * 4 + hh
        hoff = h * N
        attv = [attbuf[pl.ds(h * D + 16 * j, 16)] for j in range(8)]
        mbv = mbbuf[pl.ds(h * 16, 16)]

        def zero_row(r, _):
            for j in range(8):
                tmp_n[r, pl.ds(16 * j, 16)] = zv
            tmp_d[r, pl.ds(0, 16)] = zv
            return 0

        lax.fori_loop(0, CH, zero_row, 0, unroll=False)
        for k in range(NCH):
            for t in range(CH // 16):
                zidx[pl.ds(16 * t, 16)] = iot + (
                    s * STRIPE + CH * k + 16 * t)
            pltpu.sync_copy(tmp_n, acc_n.at[zidx])
            pltpu.sync_copy(tmp_d, acc_d.at[zidx])

        @pl.when(s == NTILES - 1)
        def _():
            pltpu.sync_copy(tmp_n, acc_n.at[tidx])
            pltpu.sync_copy(tmp_d, acc_d.at[tidx])

        plsc.subcore_barrier()

        def fetch_idx(g, q):
            pltpu.sync_copy(src_hbm.at[pl.ds(tbase + g * GROUP, GROUP)], rsrc)
            pltpu.sync_copy(dst_hbm.at[pl.ds(tbase + g * GROUP, GROUP)], rdst)
            for k in range(GROUP // 16):
                dv = rdst[pl.ds(16 * k, 16)]
                gsrc[q][pl.ds(16 * k, 16)] = rsrc[pl.ds(16 * k, 16)] + hoff
                gdst[q][pl.ds(16 * k, 16)] = dv + hoff
                sdst[q][pl.ds(16 * k, 16)] = dv
            pltpu.async_copy(xl_hbm.at[gsrc[q]], buf_l[q], seml[q])
            pltpu.async_copy(xr_hbm.at[gdst[q]], buf_r[q], semr[q])

        fetch_idx(0, 0)

        def pair_step(gp, _):
            for b in range(2):
                g = 2 * gp + b
                p, q = b, 1 - b

                @pl.when(g + 1 < NGROUPS)
                def _():
                    fetch_idx(g + 1, q)

                pltpu.make_async_copy(
                    xl_hbm.at[gsrc[p]], buf_l[p], seml[p]).wait()
                pltpu.make_async_copy(
                    xr_hbm.at[gdst[p]], buf_r[p], semr[p]).wait()

                bl, br = buf_l[p], buf_r[p]

                def sub_step(u, _):
                    for i in range(16):
                        e = u * 16 + i
                        acc = jnp.zeros((16,), _f32)
                        for j in range(8):
                            a = bl[e, pl.ds(16 * j, 16)]
                            bb = br[e, pl.ds(16 * j, 16)]
                            t = a + bb
                            t = jnp.maximum(t, 0.2 * t)
                            acc = acc + attv[j] * t
                        for sh in (1, 2, 4, 8):
                            perm = lax.iota(jnp.int32, 16) ^ sh
                            red[...] = acc
                            acc = acc + plsc.load_gather(red, [perm])
                        gid = tbase + g * GROUP + e
                        maskf = jnp.where(gid < E_TOT, 1.0, 0.0)
                        wf = jnp.exp(acc - mbv) * maskf
                        wrow[e, pl.ds(0, 16)] = wf
                        for j in range(8):
                            bl[e, pl.ds(16 * j, 16)] = (
                                wf * bl[e, pl.ds(16 * j, 16)])
                    return 0

                lax.fori_loop(0, GROUP // 16, sub_step, 0, unroll=False)
                pltpu.sync_copy(bl, acc_n.at[sdst[p]], add=True)
                pltpu.sync_copy(wrow, acc_d.at[sdst[p]], add=True)
            return 0

        lax.fori_loop(0, NGROUPS // 2, pair_step, 0, unroll=False)
        plsc.subcore_barrier()

        for k in range(NCH):
            for t in range(CH // 16):
                zidx[pl.ds(16 * t, 16)] = iot + (
                    s * STRIPE + CH * k + 16 * t)
            pltpu.async_copy(acc_n.at[zidx], tmp_n, sem).wait()
            pltpu.sync_copy(
                tmp_n, outn_hbm.at[pl.ds(h * N + s * STRIPE + CH * k, CH)])
            pltpu.async_copy(acc_d.at[zidx], tmp_d, sem).wait()
            pltpu.sync_copy(
                tmp_d, outd_hbm.at[pl.ds(h * N + s * STRIPE + CH * k, CH)])

        @pl.when(s == NTILES - 1)
        def _():
            pltpu.async_copy(acc_n.at[tidx], tmp_n, sem).wait()
            pltpu.sync_copy(
                tmp_n, outn_hbm.at[pl.ds(h * N + NTILES * STRIPE, TAIL)])
            pltpu.async_copy(acc_d.at[tidx], tmp_d, sem).wait()
            pltpu.sync_copy(
                tmp_d, outd_hbm.at[pl.ds(h * N + NTILES * STRIPE, TAIL)])

        return 0

    lax.fori_loop(0, 4, head_step, 0, unroll=False)


def _edge_phase(xl, xr, src_pad, dst_pad, att_flat, mb_flat):
    mesh = plsc.VectorSubcoreMesh(core_axis_name="c", subcore_axis_name="s")
    f = pl.kernel(
        _edge_body,
        out_type=[
            jax.ShapeDtypeStruct((H * N, D), _f32),
            jax.ShapeDtypeStruct((H * N, 16), _f32),
        ],
        mesh=mesh,
        scratch_types=[
            pltpu.VMEM_SHARED((N, D), _f32),
            pltpu.VMEM_SHARED((N, 16), _f32),
            pltpu.VMEM((GROUP, D), _f32),
            pltpu.VMEM((GROUP, D), _f32),
            pltpu.VMEM((GROUP, D), _f32),
            pltpu.VMEM((GROUP, D), _f32),
            pltpu.VMEM((GROUP, 16), _f32),
            pltpu.VMEM((GROUP,), jnp.int32),
            pltpu.VMEM((GROUP,), jnp.int32),
            pltpu.VMEM((GROUP,), jnp.int32),
            pltpu.VMEM((GROUP,), jnp.int32),
            pltpu.VMEM((GROUP,), jnp.int32),
            pltpu.VMEM((GROUP,), jnp.int32),
            pltpu.VMEM((GROUP,), jnp.int32),
            pltpu.VMEM((GROUP,), jnp.int32),
            pltpu.VMEM((NHID,), _f32),
            pltpu.VMEM((H * 16,), _f32),
            pltpu.VMEM((16,), _f32),
            pltpu.VMEM((CH, D), _f32),
            pltpu.VMEM((CH, 16), _f32),
            pltpu.VMEM((CH,), jnp.int32),
            pltpu.VMEM((TAIL,), jnp.int32),
            pltpu.SemaphoreType.DMA,
            pltpu.SemaphoreType.DMA,
            pltpu.SemaphoreType.DMA,
            pltpu.SemaphoreType.DMA,
            pltpu.SemaphoreType.DMA,
        ],
        compiler_params=pltpu.CompilerParams(needs_layout_passes=False,
                                             use_tc_tiling_on_sc=False),
    )
    return f(xl, xr, src_pad, dst_pad, att_flat, mb_flat)


def _proj_body(num_ref, den_ref, gb_ref, pw_ref, pb_ref, res_ref,
               g_ref, b_ref, o_ref, acc):
    j = pl.program_id(1)
    gat = num_ref[...] / (den_ref[:, 0:1] + 1e-16) + gb_ref[0]
    contrib = jnp.dot(gat, pw_ref[...], preferred_element_type=_f32)

    @pl.when(j == 0)
    def _():
        acc[...] = contrib

    @pl.when(j > 0)
    def _():
        acc[...] = acc[...] + contrib

    @pl.when(j == H - 1)
    def _():
        y = acc[...] + pb_ref[...] + res_ref[...]
        o_ref[...] = _ln(y, g_ref[...], b_ref[...])


def _proj_ln(num, den, gat_bias, pw, pb, res, g, b):
    return pl.pallas_call(
        _proj_body,
        grid=(NRB, H),
        in_specs=[
            pl.BlockSpec((RB, D), lambda i, j: (j * NRB + i, 0)),
            pl.BlockSpec((RB, 16), lambda i, j: (j * NRB + i, 0)),
            pl.BlockSpec((1, 1, D), lambda i, j: (j, 0, 0)),
            pl.BlockSpec((D, D), lambda i, j: (j, 0)),
            pl.BlockSpec((1, D), lambda i, j: (0, 0)),
            pl.BlockSpec((RB, D), lambda i, j: (i, 0)),
            pl.BlockSpec((1, D), lambda i, j: (0, 0)),
            pl.BlockSpec((1, D), lambda i, j: (0, 0)),
        ],
        out_specs=pl.BlockSpec((RB, D), lambda i, j: (i, 0)),
        out_shape=jax.ShapeDtypeStruct((N, D), _f32),
        scratch_shapes=[pltpu.VMEM((RB, D), _f32)],
    )(num, den, gat_bias.reshape(H, 1, D), pw, pb[None, :],
      res, g[None, :], b[None, :])


def _fc_body(h_ref, w1_ref, b1_ref, w2_ref, b2_ref, g_ref, b_ref, o_ref):
    hb = h_ref[...]
    f = _lrelu(jnp.dot(hb, w1_ref[...],
                       preferred_element_type=_f32) + b1_ref[...], 0.01)
    f = jnp.dot(f, w2_ref[...], preferred_element_type=_f32) + b2_ref[...]
    o_ref[...] = _ln(f + hb, g_ref[...], b_ref[...])


def _fc_ln(h, w1, b1, w2, b2, g, b):
    return pl.pallas_call(
        _fc_body,
        grid=(NRB,),
        in_specs=[
            pl.BlockSpec((RB, D), lambda i: (i, 0)),
            pl.BlockSpec((D, D), lambda i: (0, 0)),
            pl.BlockSpec((1, D), lambda i: (0, 0)),
            pl.BlockSpec((D, D), lambda i: (0, 0)),
            pl.BlockSpec((1, D), lambda i: (0, 0)),
            pl.BlockSpec((1, D), lambda i: (0, 0)),
            pl.BlockSpec((1, D), lambda i: (0, 0)),
        ],
        out_specs=pl.BlockSpec((RB, D), lambda i: (i, 0)),
        out_shape=jax.ShapeDtypeStruct((N, D), _f32),
    )(h, w1, b1[None, :], w2, b2[None, :], g[None, :], b[None, :])


def kernel(x, edge_index, params):
    p = params
    loops = jnp.arange(N, dtype=edge_index.dtype)
    padv = jnp.zeros((E_PAD - E_TOT,), edge_index.dtype)
    src_pad = jnp.concatenate([edge_index[0], loops, padv])
    dst_pad = jnp.concatenate([edge_index[1], loops, padv])

    h = _embed(x, p['emb_W1'], p['emb_b1'], p['emb_W2'], p['emb_b2'])
    for i in range(2):
        xl, xr, mb = _xlxr(h, p['gat%d_Wl' % i], p['gat%d_bl' % i],
                           p['gat%d_Wr' % i], p['gat%d_br' % i],
                           p['gat%d_att' % i])
        num, den = _edge_phase(xl, xr, src_pad, dst_pad,
                               p['gat%d_att' % i].reshape(-1),
                               mb.reshape(-1))
        h = _proj_ln(num, den, p['gat%d_bias' % i],
                     p['proj%d_W' % i], p['proj%d_b' % i], h,
                     p['gn%d_g' % i], p['gn%d_b' % i])
        h = _fc_ln(h, p['fc%d_W1' % i], p['fc%d_b1' % i],
                   p['fc%d_W2' % i], p['fc%d_b2' % i],
                   p['fn%d_g' % i], p['fn%d_b' % i])
    return h


# speedup vs baseline: 7.3509x; 1.0814x over previous
import functools

import jax
import jax.numpy as jnp
from jax import lax
from jax.experimental import pallas as pl
from jax.experimental.pallas import tpu as pltpu
from jax.experimental.pallas import tpu_sc as plsc

N = 10000
E = 320000
D = 128
H = 8
NHID = H * D
NTILES = 16
NSC = 2
E_TOT = E + N
GROUP = 64
NGROUPS = -(-E_TOT // (NTILES * GROUP))
NGROUPS += NGROUPS % 2
EPT = NGROUPS * GROUP
E_PAD = EPT * NTILES
STRIPE = 624
TAIL = N - NTILES * STRIPE
CH = 16
NCH = STRIPE // CH
RB = 1000
NRB = N // RB

_f32 = jnp.float32


def _lrelu(x, slope):
    return jnp.maximum(x, slope * x)


def _ln(y, g, b):
    m = jnp.mean(y, axis=-1, keepdims=True)
    v = jnp.mean((y - m) ** 2, axis=-1, keepdims=True)
    return (y - m) * lax.rsqrt(v + 1e-5) * g + b


def _embed_body(x_ref, w1_ref, b1_ref, w2_ref, b2_ref, o_ref):
    h1 = _lrelu(jnp.dot(x_ref[...], w1_ref[...],
                        preferred_element_type=_f32) + b1_ref[...], 0.01)
    o_ref[...] = jnp.dot(h1, w2_ref[...],
                         preferred_element_type=_f32) + b2_ref[...]


def _embed(x, w1, b1, w2, b2):
    return pl.pallas_call(
        _embed_body,
        grid=(NRB,),
        in_specs=[
            pl.BlockSpec((RB, D), lambda i: (i, 0)),
            pl.BlockSpec((D, D), lambda i: (0, 0)),
            pl.BlockSpec((1, D), lambda i: (0, 0)),
            pl.BlockSpec((D, D), lambda i: (0, 0)),
            pl.BlockSpec((1, D), lambda i: (0, 0)),
        ],
        out_specs=pl.BlockSpec((RB, D), lambda i: (i, 0)),
        out_shape=jax.ShapeDtypeStruct((N, D), _f32),
    )(x, w1, b1[None, :], w2, b2[None, :])


def _xlxr_body(h_ref, wl_ref, bl_ref, wr_ref, br_ref, att_ref,
               xl_ref, xr_ref, m_ref, sm):
    i = pl.program_id(1)
    xlb = jnp.dot(h_ref[...], wl_ref[...],
                  preferred_element_type=_f32) + bl_ref[0]
    xrb = jnp.dot(h_ref[...], wr_ref[...],
                  preferred_element_type=_f32) + br_ref[0]
    xl_ref[...] = xlb
    xr_ref[...] = xrb
    attr = jnp.abs(att_ref[0])
    pmax = jnp.max(jnp.sum(jnp.abs(xlb) * attr, axis=1))
    qmax = jnp.max(jnp.sum(jnp.abs(xrb) * attr, axis=1))

    @pl.when(i == 0)
    def _():
        sm[0] = pmax
        sm[1] = qmax

    @pl.when(i > 0)
    def _():
        sm[0] = jnp.maximum(sm[0], pmax)
        sm[1] = jnp.maximum(sm[1], qmax)

    m_ref[...] = jnp.full((1, 1, 16), sm[0] + sm[1], _f32)


def _xlxr(h, wl, bl, wr, br, att):
    return pl.pallas_call(
        _xlxr_body,
        grid=(H, NRB),
        in_specs=[
            pl.BlockSpec((RB, D), lambda j, i: (i, 0)),
            pl.BlockSpec((D, D), lambda j, i: (0, j)),
            pl.BlockSpec((1, 1, D), lambda j, i: (j, 0, 0)),
            pl.BlockSpec((D, D), lambda j, i: (0, j)),
            pl.BlockSpec((1, 1, D), lambda j, i: (j, 0, 0)),
            pl.BlockSpec((1, 1, D), lambda j, i: (j, 0, 0)),
        ],
        out_specs=[
            pl.BlockSpec((RB, D), lambda j, i: (j * NRB + i, 0)),
            pl.BlockSpec((RB, D), lambda j, i: (j * NRB + i, 0)),
            pl.BlockSpec((1, 1, 16), lambda j, i: (j, 0, 0)),
        ],
        out_shape=[
            jax.ShapeDtypeStruct((H * N, D), _f32),
            jax.ShapeDtypeStruct((H * N, D), _f32),
            jax.ShapeDtypeStruct((H, 1, 16), _f32),
        ],
        scratch_shapes=[pltpu.SMEM((2,), _f32)],
    )(h, wl, bl.reshape(H, 1, D), wr, br.reshape(H, 1, D),
      att.reshape(H, 1, D))


def _edge_body(xl_hbm, xr_hbm, src_hbm, dst_hbm, att_hbm, mb_hbm,
               outn_hbm, outd_hbm,
               acc_n, acc_d, buf_l0, buf_l1, buf_r0, buf_r1, wrow,
               rsrc, rdst, gsrc0, gsrc1, gdst0, gdst1, sdst0, sdst1,
               attbuf, mbbuf, red, tmp_n, tmp_d, zidx, tidx,
               sem, semd, seml0, seml1, semr0, semr1):
    c = lax.axis_index("c")
    s = lax.axis_index("s")
    tbase = s * EPT
    buf_l = (buf_l0, buf_l1)
    buf_r = (buf_r0, buf_r1)
    gsrc = (gsrc0, gsrc1)
    gdst = (gdst0, gdst1)
    sdst = (sdst0, sdst1)
    seml = (seml0, seml1)
    semr = (semr0, semr1)

    pltpu.sync_copy(att_hbm, attbuf)
    pltpu.sync_copy(mb_hbm, mbbuf)

    zv = jnp.zeros((16,), _f32)
    iot = lax.iota(jnp.int32, 16)
    tidx[...] = iot + (NTILES * STRIPE)

    def head_step(hh, _):
        h = c * 4 + hh
        hoff = h * N
        attv = [attbuf[pl.ds(h * D + 16 * j, 16)] for j in range(8)]
        mbv = mbbuf[pl.ds(h * 16, 16)]

        def zero_row(r, _):
            for j in range(8):
                tmp_n[r, pl.ds(16 * j, 16)] = zv
            tmp_d[r, pl.ds(0, 16)] = zv
            return 0

        lax.fori_loop(0, CH, zero_row, 0, unroll=False)
        for k in range(NCH):
            for t in range(CH // 16):
                zidx[pl.ds(16 * t, 16)] = iot + (
                    s * STRIPE + CH * k + 16 * t)
            pltpu.sync_copy(tmp_n, acc_n.at[zidx])
            pltpu.sync_copy(tmp_d, acc_d.at[zidx])

        @pl.when(s == NTILES - 1)
        def _():
            pltpu.sync_copy(tmp_n, acc_n.at[tidx])
            pltpu.sync_copy(tmp_d, acc_d.at[tidx])

        plsc.subcore_barrier()

        def fetch_idx(g, q):
            ci = pltpu.async_copy(
                src_hbm.at[pl.ds(tbase + g * GROUP, GROUP)], rsrc, seml[q])
            cj = pltpu.async_copy(
                dst_hbm.at[pl.ds(tbase + g * GROUP, GROUP)], rdst, semr[q])
            ci.wait()
            cj.wait()
            for k in range(GROUP // 16):
                dv = rdst[pl.ds(16 * k, 16)]
                gsrc[q][pl.ds(16 * k, 16)] = rsrc[pl.ds(16 * k, 16)] + hoff
                gdst[q][pl.ds(16 * k, 16)] = dv + hoff
                sdst[q][pl.ds(16 * k, 16)] = dv
            pltpu.async_copy(xl_hbm.at[gsrc[q]], buf_l[q], seml[q])
            pltpu.async_copy(xr_hbm.at[gdst[q]], buf_r[q], semr[q])

        fetch_idx(0, 0)

        def pair_step(gp, _):
            for b in range(2):
                g = 2 * gp + b
                p, q = b, 1 - b

                @pl.when(g + 1 < NGROUPS)
                def _():
                    fetch_idx(g + 1, q)

                pltpu.make_async_copy(
                    xl_hbm.at[gsrc[p]], buf_l[p], seml[p]).wait()
                pltpu.make_async_copy(
                    xr_hbm.at[gdst[p]], buf_r[p], semr[p]).wait()

                bl, br = buf_l[p], buf_r[p]

                def sub_step(u, _):
                    for i in range(16):
                        e = u * 16 + i
                        acc = jnp.zeros((16,), _f32)
                        for j in range(8):
                            a = bl[e, pl.ds(16 * j, 16)]
                            bb = br[e, pl.ds(16 * j, 16)]
                            t = a + bb
                            t = jnp.maximum(t, 0.2 * t)
                            acc = acc + attv[j] * t
                        for sh in (1, 2, 4, 8):
                            perm = lax.iota(jnp.int32, 16) ^ sh
                            red[...] = acc
                            acc = acc + plsc.load_gather(red, [perm])
                        gid = tbase + g * GROUP + e
                        maskf = jnp.where(gid < E_TOT, 1.0, 0.0)
                        wf = jnp.exp(acc - mbv) * maskf
                        wrow[e, pl.ds(0, 16)] = wf
                        for j in range(8):
                            bl[e, pl.ds(16 * j, 16)] = (
                                wf * bl[e, pl.ds(16 * j, 16)])
                    return 0

                lax.fori_loop(0, GROUP // 16, sub_step, 0, unroll=False)
                cn = pltpu.async_copy(bl, acc_n.at[sdst[p]], sem,
                                      add=True)
                cd = pltpu.async_copy(wrow, acc_d.at[sdst[p]], semd,
                                      add=True)
                cn.wait()
                cd.wait()
            return 0

        lax.fori_loop(0, NGROUPS // 2, pair_step, 0, unroll=False)
        plsc.subcore_barrier()

        for k in range(NCH):
            for t in range(CH // 16):
                zidx[pl.ds(16 * t, 16)] = iot + (
                    s * STRIPE + CH * k + 16 * t)
            pltpu.async_copy(acc_n.at[zidx], tmp_n, sem).wait()
            pltpu.sync_copy(
                tmp_n, outn_hbm.at[pl.ds(h * N + s * STRIPE + CH * k, CH)])
            pltpu.async_copy(acc_d.at[zidx], tmp_d, sem).wait()
            pltpu.sync_copy(
                tmp_d, outd_hbm.at[pl.ds(h * N + s * STRIPE + CH * k, CH)])

        @pl.when(s == NTILES - 1)
        def _():
            pltpu.async_copy(acc_n.at[tidx], tmp_n, sem).wait()
            pltpu.sync_copy(
                tmp_n, outn_hbm.at[pl.ds(h * N + NTILES * STRIPE, TAIL)])
            pltpu.async_copy(acc_d.at[tidx], tmp_d, sem).wait()
            pltpu.sync_copy(
                tmp_d, outd_hbm.at[pl.ds(h * N + NTILES * STRIPE, TAIL)])

        return 0

    lax.fori_loop(0, 4, head_step, 0, unroll=False)


def _edge_phase(xl, xr, src_pad, dst_pad, att_flat, mb_flat):
    mesh = plsc.VectorSubcoreMesh(core_axis_name="c", subcore_axis_name="s")
    f = pl.kernel(
        _edge_body,
        out_type=[
            jax.ShapeDtypeStruct((H * N, D), _f32),
            jax.ShapeDtypeStruct((H * N, 16), _f32),
        ],
        mesh=mesh,
        scratch_types=[
            pltpu.VMEM_SHARED((N, D), _f32),
            pltpu.VMEM_SHARED((N, 16), _f32),
            pltpu.VMEM((GROUP, D), _f32),
            pltpu.VMEM((GROUP, D), _f32),
            pltpu.VMEM((GROUP, D), _f32),
            pltpu.VMEM((GROUP, D), _f32),
            pltpu.VMEM((GROUP, 16), _f32),
            pltpu.VMEM((GROUP,), jnp.int32),
            pltpu.VMEM((GROUP,), jnp.int32),
            pltpu.VMEM((GROUP,), jnp.int32),
            pltpu.VMEM((GROUP,), jnp.int32),
            pltpu.VMEM((GROUP,), jnp.int32),
            pltpu.VMEM((GROUP,), jnp.int32),
            pltpu.VMEM((GROUP,), jnp.int32),
            pltpu.VMEM((GROUP,), jnp.int32),
            pltpu.VMEM((NHID,), _f32),
            pltpu.VMEM((H * 16,), _f32),
            pltpu.VMEM((16,), _f32),
            pltpu.VMEM((CH, D), _f32),
            pltpu.VMEM((CH, 16), _f32),
            pltpu.VMEM((CH,), jnp.int32),
            pltpu.VMEM((TAIL,), jnp.int32),
            pltpu.SemaphoreType.DMA,
            pltpu.SemaphoreType.DMA,
            pltpu.SemaphoreType.DMA,
            pltpu.SemaphoreType.DMA,
            pltpu.SemaphoreType.DMA,
            pltpu.SemaphoreType.DMA,
        ],
        compiler_params=pltpu.CompilerParams(needs_layout_passes=False,
                                             use_tc_tiling_on_sc=False),
    )
    return f(xl, xr, src_pad, dst_pad, att_flat, mb_flat)


def _proj_body(num_ref, den_ref, gb_ref, pw_ref, pb_ref, res_ref,
               g_ref, b_ref, o_ref, acc):
    j = pl.program_id(1)
    gat = num_ref[...] / (den_ref[:, 0:1] + 1e-16) + gb_ref[0]
    contrib = jnp.dot(gat, pw_ref[...], preferred_element_type=_f32)

    @pl.when(j == 0)
    def _():
        acc[...] = contrib

    @pl.when(j > 0)
    def _():
        acc[...] = acc[...] + contrib

    @pl.when(j == H - 1)
    def _():
        y = acc[...] + pb_ref[...] + res_ref[...]
        o_ref[...] = _ln(y, g_ref[...], b_ref[...])


def _proj_ln(num, den, gat_bias, pw, pb, res, g, b):
    return pl.pallas_call(
        _proj_body,
        grid=(NRB, H),
        in_specs=[
            pl.BlockSpec((RB, D), lambda i, j: (j * NRB + i, 0)),
            pl.BlockSpec((RB, 16), lambda i, j: (j * NRB + i, 0)),
            pl.BlockSpec((1, 1, D), lambda i, j: (j, 0, 0)),
            pl.BlockSpec((D, D), lambda i, j: (j, 0)),
            pl.BlockSpec((1, D), lambda i, j: (0, 0)),
            pl.BlockSpec((RB, D), lambda i, j: (i, 0)),
            pl.BlockSpec((1, D), lambda i, j: (0, 0)),
            pl.BlockSpec((1, D), lambda i, j: (0, 0)),
        ],
        out_specs=pl.BlockSpec((RB, D), lambda i, j: (i, 0)),
        out_shape=jax.ShapeDtypeStruct((N, D), _f32),
        scratch_shapes=[pltpu.VMEM((RB, D), _f32)],
    )(num, den, gat_bias.reshape(H, 1, D), pw, pb[None, :],
      res, g[None, :], b[None, :])


def _fc_body(h_ref, w1_ref, b1_ref, w2_ref, b2_ref, g_ref, b_ref, o_ref):
    hb = h_ref[...]
    f = _lrelu(jnp.dot(hb, w1_ref[...],
                       preferred_element_type=_f32) + b1_ref[...], 0.01)
    f = jnp.dot(f, w2_ref[...], preferred_element_type=_f32) + b2_ref[...]
    o_ref[...] = _ln(f + hb, g_ref[...], b_ref[...])


def _fc_ln(h, w1, b1, w2, b2, g, b):
    return pl.pallas_call(
        _fc_body,
        grid=(NRB,),
        in_specs=[
            pl.BlockSpec((RB, D), lambda i: (i, 0)),
            pl.BlockSpec((D, D), lambda i: (0, 0)),
            pl.BlockSpec((1, D), lambda i: (0, 0)),
            pl.BlockSpec((D, D), lambda i: (0, 0)),
            pl.BlockSpec((1, D), lambda i: (0, 0)),
            pl.BlockSpec((1, D), lambda i: (0, 0)),
            pl.BlockSpec((1, D), lambda i: (0, 0)),
        ],
        out_specs=pl.BlockSpec((RB, D), lambda i: (i, 0)),
        out_shape=jax.ShapeDtypeStruct((N, D), _f32),
    )(h, w1, b1[None, :], w2, b2[None, :], g[None, :], b[None, :])


def kernel(x, edge_index, params):
    p = params
    loops = jnp.arange(N, dtype=edge_index.dtype)
    padv = jnp.zeros((E_PAD - E_TOT,), edge_index.dtype)
    src_pad = jnp.concatenate([edge_index[0], loops, padv])
    dst_pad = jnp.concatenate([edge_index[1], loops, padv])

    h = _embed(x, p['emb_W1'], p['emb_b1'], p['emb_W2'], p['emb_b2'])
    for i in range(2):
        xl, xr, mb = _xlxr(h, p['gat%d_Wl' % i], p['gat%d_bl' % i],
                           p['gat%d_Wr' % i], p['gat%d_br' % i],
                           p['gat%d_att' % i])
        num, den = _edge_phase(xl, xr, src_pad, dst_pad,
                               p['gat%d_att' % i].reshape(-1),
                               mb.reshape(-1))
        h = _proj_ln(num, den, p['gat%d_bias' % i],
                     p['proj%d_W' % i], p['proj%d_b' % i], h,
                     p['gn%d_g' % i], p['gn%d_b' % i])
        h = _fc_ln(h, p['fc%d_W1' % i], p['fc%d_b1' % i],
                   p['fc%d_W2' % i], p['fc%d_b2' % i],
                   p['fn%d_g' % i], p['fn%d_b' % i])
    return h


# speedup vs baseline: 7.3518x; 1.0001x over previous
import jax
import jax.numpy as jnp
from jax import lax
from jax.experimental import pallas as pl
from jax.experimental.pallas import tpu as pltpu
from jax.experimental.pallas import tpu_sc as plsc

N = 10000
E = 320000
D = 128
H = 8
NHID = H * D
NTILES = 16
NSC = 2
E_TOT = E + N
GROUP = 64
NGROUPS = -(-E_TOT // (NTILES * GROUP))
NGROUPS += NGROUPS % 2
EPT = NGROUPS * GROUP
E_PAD = EPT * NTILES
STRIPE = 624
TAIL = N - NTILES * STRIPE
CH = 16
NCH = STRIPE // CH
RB = 1000
NRB = N // RB

_f32 = jnp.float32


def _lrelu(x, slope):
    return jnp.maximum(x, slope * x)


def _ln(y, g, b):
    m = jnp.mean(y, axis=-1, keepdims=True)
    v = jnp.mean((y - m) ** 2, axis=-1, keepdims=True)
    return (y - m) * lax.rsqrt(v + 1e-5) * g + b


def _embed_body(x_ref, w1_ref, b1_ref, w2_ref, b2_ref, o_ref):
    h1 = _lrelu(jnp.dot(x_ref[...], w1_ref[...],
                        preferred_element_type=_f32) + b1_ref[...], 0.01)
    o_ref[...] = jnp.dot(h1, w2_ref[...],
                         preferred_element_type=_f32) + b2_ref[...]


def _embed(x, w1, b1, w2, b2):
    return pl.pallas_call(
        _embed_body,
        grid=(NRB,),
        in_specs=[
            pl.BlockSpec((RB, D), lambda i: (i, 0)),
            pl.BlockSpec((D, D), lambda i: (0, 0)),
            pl.BlockSpec((1, D), lambda i: (0, 0)),
            pl.BlockSpec((D, D), lambda i: (0, 0)),
            pl.BlockSpec((1, D), lambda i: (0, 0)),
        ],
        out_specs=pl.BlockSpec((RB, D), lambda i: (i, 0)),
        out_shape=jax.ShapeDtypeStruct((N, D), _f32),
    )(x, w1, b1[None, :], w2, b2[None, :])


def _xlxr_body(h_ref, wl_ref, bl_ref, wr_ref, br_ref, att_ref,
               xl_ref, xr_ref, m_ref, sm):
    i = pl.program_id(1)
    xlb = jnp.dot(h_ref[...], wl_ref[...],
                  preferred_element_type=_f32) + bl_ref[0]
    xrb = jnp.dot(h_ref[...], wr_ref[...],
                  preferred_element_type=_f32) + br_ref[0]
    xl_ref[...] = xlb
    xr_ref[...] = xrb
    attr = jnp.abs(att_ref[0])
    pmax = jnp.max(jnp.sum(jnp.abs(xlb) * attr, axis=1))
    qmax = jnp.max(jnp.sum(jnp.abs(xrb) * attr, axis=1))

    @pl.when(i == 0)
    def _():
        sm[0] = pmax
        sm[1] = qmax

    @pl.when(i > 0)
    def _():
        sm[0] = jnp.maximum(sm[0], pmax)
        sm[1] = jnp.maximum(sm[1], qmax)

    m_ref[...] = jnp.full((1, 1, 16), sm[0] + sm[1], _f32)


def _xlxr(h, wl, bl, wr, br, att):
    return pl.pallas_call(
        _xlxr_body,
        grid=(H, NRB),
        in_specs=[
            pl.BlockSpec((RB, D), lambda j, i: (i, 0)),
            pl.BlockSpec((D, D), lambda j, i: (0, j)),
            pl.BlockSpec((1, 1, D), lambda j, i: (j, 0, 0)),
            pl.BlockSpec((D, D), lambda j, i: (0, j)),
            pl.BlockSpec((1, 1, D), lambda j, i: (j, 0, 0)),
            pl.BlockSpec((1, 1, D), lambda j, i: (j, 0, 0)),
        ],
        out_specs=[
            pl.BlockSpec((RB, D), lambda j, i: (j * NRB + i, 0)),
            pl.BlockSpec((RB, D), lambda j, i: (j * NRB + i, 0)),
            pl.BlockSpec((1, 1, 16), lambda j, i: (j, 0, 0)),
        ],
        out_shape=[
            jax.ShapeDtypeStruct((H * N, D), _f32),
            jax.ShapeDtypeStruct((H * N, D), _f32),
            jax.ShapeDtypeStruct((H, 1, 16), _f32),
        ],
        scratch_shapes=[pltpu.SMEM((2,), _f32)],
    )(h, wl, bl.reshape(H, 1, D), wr, br.reshape(H, 1, D),
      att.reshape(H, 1, D))


def _edge_body(xl_hbm, xr_hbm, src_hbm, dst_hbm, att_hbm, mb_hbm,
               outn_hbm, outd_hbm,
               acc_n, acc_d, buf_l0, buf_l1, buf_r0, buf_r1, wrow,
               rsrc, rdst, gsrc0, gsrc1, gdst0, gdst1, sdst0, sdst1,
               attbuf, mbbuf, red, tmp_n, tmp_d, zidx, tidx,
               sem, semd, seml0, seml1, semr0, semr1):
    c = lax.axis_index("c")
    s = lax.axis_index("s")
    tbase = s * EPT
    buf_l = (buf_l0, buf_l1)
    buf_r = (buf_r0, buf_r1)
    gsrc = (gsrc0, gsrc1)
    gdst = (gdst0, gdst1)
    sdst = (sdst0, sdst1)
    seml = (seml0, seml1)
    semr = (semr0, semr1)

    pltpu.sync_copy(att_hbm, attbuf)
    pltpu.sync_copy(mb_hbm, mbbuf)

    zv = jnp.zeros((16,), _f32)
    iot = lax.iota(jnp.int32, 16)
    tidx[...] = iot + (NTILES * STRIPE)

    def head_step(hh, _):
        h = c * 4 + hh
        hoff = h * N
        attv = [attbuf[pl.ds(h * D + 16 * j, 16)] for j in range(8)]
        mbv = mbbuf[pl.ds(h * 16, 16)]

        def zero_row(r, _):
            for j in range(8):
                tmp_n[r, pl.ds(16 * j, 16)] = zv
            tmp_d[r, pl.ds(0, 16)] = zv
            return 0

        lax.fori_loop(0, CH, zero_row, 0, unroll=False)
        for k in range(NCH):
            for t in range(CH // 16):
                zidx[pl.ds(16 * t, 16)] = iot + (
                    s * STRIPE + CH * k + 16 * t)
            pltpu.sync_copy(tmp_n, acc_n.at[zidx])
            pltpu.sync_copy(tmp_d, acc_d.at[zidx])

        @pl.when(s == NTILES - 1)
        def _():
            pltpu.sync_copy(tmp_n, acc_n.at[tidx])
            pltpu.sync_copy(tmp_d, acc_d.at[tidx])

        plsc.subcore_barrier()

        def fetch_idx(g, q):
            ci = pltpu.async_copy(
                src_hbm.at[pl.ds(tbase + g * GROUP, GROUP)], rsrc, seml[q])
            cj = pltpu.async_copy(
                dst_hbm.at[pl.ds(tbase + g * GROUP, GROUP)], rdst, semr[q])
            ci.wait()
            cj.wait()
            for k in range(GROUP // 16):
                dv = rdst[pl.ds(16 * k, 16)]
                gsrc[q][pl.ds(16 * k, 16)] = rsrc[pl.ds(16 * k, 16)] + hoff
                gdst[q][pl.ds(16 * k, 16)] = dv + hoff
                sdst[q][pl.ds(16 * k, 16)] = dv
            pltpu.async_copy(xl_hbm.at[gsrc[q]], buf_l[q], seml[q])
            pltpu.async_copy(xr_hbm.at[gdst[q]], buf_r[q], semr[q])

        fetch_idx(0, 0)

        def pair_step(gp, _):
            for b in range(2):
                g = 2 * gp + b
                p, q = b, 1 - b

                @pl.when(g + 1 < NGROUPS)
                def _():
                    fetch_idx(g + 1, q)

                pltpu.make_async_copy(
                    xl_hbm.at[gsrc[p]], buf_l[p], seml[p]).wait()
                pltpu.make_async_copy(
                    xr_hbm.at[gdst[p]], buf_r[p], semr[p]).wait()

                bl, br = buf_l[p], buf_r[p]

                def sub_step(u, _):
                    for i in range(16):
                        e = u * 16 + i
                        acc = jnp.zeros((16,), _f32)
                        for j in range(8):
                            a = bl[e, pl.ds(16 * j, 16)]
                            bb = br[e, pl.ds(16 * j, 16)]
                            t = a + bb
                            t = jnp.maximum(t, 0.2 * t)
                            acc = acc + attv[j] * t
                        for sh in (1, 2, 4, 8):
                            perm = lax.iota(jnp.int32, 16) ^ sh
                            red[...] = acc
                            acc = acc + plsc.load_gather(red, [perm])
                        gid = tbase + g * GROUP + e
                        maskf = jnp.where(gid < E_TOT, 1.0, 0.0)
                        wf = jnp.exp(acc - mbv) * maskf
                        wrow[e, pl.ds(0, 16)] = wf
                        for j in range(8):
                            bl[e, pl.ds(16 * j, 16)] = (
                                wf * bl[e, pl.ds(16 * j, 16)])
                    return 0

                lax.fori_loop(0, GROUP // 16, sub_step, 0, unroll=False)
                cn = pltpu.async_copy(bl, acc_n.at[sdst[p]], sem,
                                      add=True)
                cd = pltpu.async_copy(wrow, acc_d.at[sdst[p]], semd,
                                      add=True)
                cn.wait()
                cd.wait()
            return 0

        lax.fori_loop(0, NGROUPS // 2, pair_step, 0, unroll=False)
        plsc.subcore_barrier()

        for k in range(NCH):
            for t in range(CH // 16):
                zidx[pl.ds(16 * t, 16)] = iot + (
                    s * STRIPE + CH * k + 16 * t)
            pltpu.async_copy(acc_n.at[zidx], tmp_n, sem).wait()
            pltpu.sync_copy(
                tmp_n, outn_hbm.at[pl.ds(h * N + s * STRIPE + CH * k, CH)])
            pltpu.async_copy(acc_d.at[zidx], tmp_d, sem).wait()
            pltpu.sync_copy(
                tmp_d, outd_hbm.at[pl.ds(h * N + s * STRIPE + CH * k, CH)])

        @pl.when(s == NTILES - 1)
        def _():
            pltpu.async_copy(acc_n.at[tidx], tmp_n, sem).wait()
            pltpu.sync_copy(
                tmp_n, outn_hbm.at[pl.ds(h * N + NTILES * STRIPE, TAIL)])
            pltpu.async_copy(acc_d.at[tidx], tmp_d, sem).wait()
            pltpu.sync_copy(
                tmp_d, outd_hbm.at[pl.ds(h * N + NTILES * STRIPE, TAIL)])

        return 0

    lax.fori_loop(0, 4, head_step, 0, unroll=False)


def _edge_phase(xl, xr, src_pad, dst_pad, att_flat, mb_flat):
    mesh = plsc.VectorSubcoreMesh(core_axis_name="c", subcore_axis_name="s")
    f = pl.kernel(
        _edge_body,
        out_type=[
            jax.ShapeDtypeStruct((H * N, D), _f32),
            jax.ShapeDtypeStruct((H * N, 16), _f32),
        ],
        mesh=mesh,
        scratch_types=[
            pltpu.VMEM_SHARED((N, D), _f32),
            pltpu.VMEM_SHARED((N, 16), _f32),
            pltpu.VMEM((GROUP, D), _f32),
            pltpu.VMEM((GROUP, D), _f32),
            pltpu.VMEM((GROUP, D), _f32),
            pltpu.VMEM((GROUP, D), _f32),
            pltpu.VMEM((GROUP, 16), _f32),
            pltpu.VMEM((GROUP,), jnp.int32),
            pltpu.VMEM((GROUP,), jnp.int32),
            pltpu.VMEM((GROUP,), jnp.int32),
            pltpu.VMEM((GROUP,), jnp.int32),
            pltpu.VMEM((GROUP,), jnp.int32),
            pltpu.VMEM((GROUP,), jnp.int32),
            pltpu.VMEM((GROUP,), jnp.int32),
            pltpu.VMEM((GROUP,), jnp.int32),
            pltpu.VMEM((NHID,), _f32),
            pltpu.VMEM((H * 16,), _f32),
            pltpu.VMEM((16,), _f32),
            pltpu.VMEM((CH, D), _f32),
            pltpu.VMEM((CH, 16), _f32),
            pltpu.VMEM((CH,), jnp.int32),
            pltpu.VMEM((TAIL,), jnp.int32),
            pltpu.SemaphoreType.DMA,
            pltpu.SemaphoreType.DMA,
            pltpu.SemaphoreType.DMA,
            pltpu.SemaphoreType.DMA,
            pltpu.SemaphoreType.DMA,
            pltpu.SemaphoreType.DMA,
        ],
        compiler_params=pltpu.CompilerParams(needs_layout_passes=False,
                                             use_tc_tiling_on_sc=False),
    )
    return f(xl, xr, src_pad, dst_pad, att_flat, mb_flat)


def _proj_body(num_ref, den_ref, gb_ref, pw_ref, pb_ref, res_ref,
               g_ref, b_ref, o_ref, acc):
    j = pl.program_id(1)
    gat = num_ref[...] / (den_ref[:, 0:1] + 1e-16) + gb_ref[0]
    contrib = jnp.dot(gat, pw_ref[...], preferred_element_type=_f32)

    @pl.when(j == 0)
    def _():
        acc[...] = contrib

    @pl.when(j > 0)
    def _():
        acc[...] = acc[...] + contrib

    @pl.when(j == H - 1)
    def _():
        y = acc[...] + pb_ref[...] + res_ref[...]
        o_ref[...] = _ln(y, g_ref[...], b_ref[...])


def _proj_ln(num, den, gat_bias, pw, pb, res, g, b):
    return pl.pallas_call(
        _proj_body,
        grid=(NRB, H),
        in_specs=[
            pl.BlockSpec((RB, D), lambda i, j: (j * NRB + i, 0)),
            pl.BlockSpec((RB, 16), lambda i, j: (j * NRB + i, 0)),
            pl.BlockSpec((1, 1, D), lambda i, j: (j, 0, 0)),
            pl.BlockSpec((D, D), lambda i, j: (j, 0)),
            pl.BlockSpec((1, D), lambda i, j: (0, 0)),
            pl.BlockSpec((RB, D), lambda i, j: (i, 0)),
            pl.BlockSpec((1, D), lambda i, j: (0, 0)),
            pl.BlockSpec((1, D), lambda i, j: (0, 0)),
        ],
        out_specs=pl.BlockSpec((RB, D), lambda i, j: (i, 0)),
        out_shape=jax.ShapeDtypeStruct((N, D), _f32),
        scratch_shapes=[pltpu.VMEM((RB, D), _f32)],
    )(num, den, gat_bias.reshape(H, 1, D), pw, pb[None, :],
      res, g[None, :], b[None, :])


def _fc_body(h_ref, w1_ref, b1_ref, w2_ref, b2_ref, g_ref, b_ref, o_ref):
    hb = h_ref[...]
    f = _lrelu(jnp.dot(hb, w1_ref[...],
                       preferred_element_type=_f32) + b1_ref[...], 0.01)
    f = jnp.dot(f, w2_ref[...], preferred_element_type=_f32) + b2_ref[...]
    o_ref[...] = _ln(f + hb, g_ref[...], b_ref[...])


def _fc_ln(h, w1, b1, w2, b2, g, b):
    return pl.pallas_call(
        _fc_body,
        grid=(NRB,),
        in_specs=[
            pl.BlockSpec((RB, D), lambda i: (i, 0)),
            pl.BlockSpec((D, D), lambda i: (0, 0)),
            pl.BlockSpec((1, D), lambda i: (0, 0)),
            pl.BlockSpec((D, D), lambda i: (0, 0)),
            pl.BlockSpec((1, D), lambda i: (0, 0)),
            pl.BlockSpec((1, D), lambda i: (0, 0)),
            pl.BlockSpec((1, D), lambda i: (0, 0)),
        ],
        out_specs=pl.BlockSpec((RB, D), lambda i: (i, 0)),
        out_shape=jax.ShapeDtypeStruct((N, D), _f32),
    )(h, w1, b1[None, :], w2, b2[None, :], g[None, :], b[None, :])


def kernel(x, edge_index, params):
    p = params
    loops = jnp.arange(N, dtype=edge_index.dtype)
    padv = jnp.zeros((E_PAD - E_TOT,), edge_index.dtype)
    src_pad = jnp.concatenate([edge_index[0], loops, padv])
    dst_pad = jnp.concatenate([edge_index[1], loops, padv])

    h = _embed(x, p['emb_W1'], p['emb_b1'], p['emb_W2'], p['emb_b2'])
    for i in range(2):
        xl, xr, mb = _xlxr(h, p['gat%d_Wl' % i], p['gat%d_bl' % i],
                           p['gat%d_Wr' % i], p['gat%d_br' % i],
                           p['gat%d_att' % i])
        num, den = _edge_phase(xl, xr, src_pad, dst_pad,
                               p['gat%d_att' % i].reshape(-1),
                               mb.reshape(-1))
        h = _proj_ln(num, den, p['gat%d_bias' % i],
                     p['proj%d_W' % i], p['proj%d_b' % i], h,
                     p['gn%d_g' % i], p['gn%d_b' % i])
        h = _fc_ln(h, p['fc%d_W1' % i], p['fc%d_b1' % i],
                   p['fc%d_W2' % i], p['fc%d_b2' % i],
                   p['fn%d_g' % i], p['fn%d_b' % i])
    return h
